# Initial kernel scaffold; baseline (speedup 1.0000x reference)
import jax
import jax.numpy as jnp
from jax.experimental import pallas as pl


def kernel(x, y, index):
    raise NotImplementedError("write your pallas kernel here")


# speedup vs baseline: 2.2040x; 2.2040x over previous
import functools

import jax
import jax.numpy as jnp
from jax import lax
from jax.experimental import pallas as pl
from jax.experimental.pallas import tpu as pltpu
from jax.experimental.pallas import tpu_sc as plsc

_M = 1000000
_D = 16
_B = 16384

_LANES = 128
_XROWS = _M * _D // _LANES
_YROWS = _B * _D // _LANES
_XBLK = 5000

_NC = 2
_NS = 16
_NW = _NC * _NS
_BPW = _B // _NW
_CH = 128
_NCH = _BPW // _CH


def _dbl_body(a_ref, o_ref):
    o_ref[...] = a_ref[...] + a_ref[...]


_tc_dbl_x = pl.pallas_call(
    _dbl_body,
    grid=(_XROWS // _XBLK,),
    in_specs=[pl.BlockSpec((_XBLK, _LANES), lambda i: (i, 0))],
    out_specs=pl.BlockSpec((_XBLK, _LANES), lambda i: (i, 0)),
    out_shape=jax.ShapeDtypeStruct((_XROWS, _LANES), jnp.float32),
)

_tc_dbl_y = pl.pallas_call(
    _dbl_body,
    out_shape=jax.ShapeDtypeStruct((_YROWS, _LANES), jnp.float32),
)


def _sc_scatter_body(yy_hbm, idx_hbm, out_ref, idx_v, rows_v, sem):
    wid = lax.axis_index("s") * _NC + lax.axis_index("c")
    pltpu.sync_copy(idx_hbm.at[pl.ds(wid * _NCH, _NCH)], idx_v)
    pltpu.sync_copy(yy_hbm.at[pl.ds(wid * _BPW, _BPW)], rows_v)
    copies = [
        pltpu.async_copy(
            rows_v.at[pl.ds(j * _CH, _CH)], out_ref.at[idx_v.at[j]], sem
        )
        for j in range(_NCH)
    ]
    for c in copies:
        c.wait()


_sc_scatter = pl.kernel(
    _sc_scatter_body,
    out_type=(),
    mesh=plsc.VectorSubcoreMesh(core_axis_name="c", subcore_axis_name="s"),
    compiler_params=pltpu.CompilerParams(use_tc_tiling_on_sc=False),
    scratch_types=[
        pltpu.VMEM((_NCH, _CH), jnp.int32),
        pltpu.VMEM((_BPW, _D), jnp.float32),
        pltpu.SemaphoreType.DMA,
    ],
)


def kernel(x, y, index):
    xx = _tc_dbl_x(x.reshape(_XROWS, _LANES)).reshape(_M, _D)
    yy = _tc_dbl_y(y.reshape(_YROWS, _LANES)).reshape(_B, _D)
    idx2 = index.reshape(_NW * _NCH, _CH)
    out_ref = jax.new_ref(xx)
    _sc_scatter(yy, idx2, out_ref)
    return jax.freeze(out_ref)
